# baseline (speedup 1.0000x reference)
import jax
import jax.numpy as jnp
from jax import lax
from jax.experimental import pallas as pl
from jax.experimental.pallas import tpu as pltpu
from jax.experimental.pallas import tpu_sc as plsc

CROP = 7
NSAMP = CROP * CROP
NSAMP_PAD = 56
C = 256
PATCH = 4 * C
N_ROI = 1000
NW = 32
ROIS_PER_W = 32
OUT_ROWS = NSAMP * C // 128


def _sc_body(table, prop, out, prop_v, idx_v, w_v, rows_v, out_v, sem):
    wid = lax.axis_index("s") * 2 + lax.axis_index("c")
    base_roi = wid * ROIS_PER_W
    pltpu.sync_copy(prop.at[pl.ds(base_roi * 7, ROIS_PER_W * 7)], prop_v)

    iota = lax.iota(jnp.int32, 16)
    zeros16 = jnp.zeros((16,), dtype=jnp.int32)
    for b in range(2):
        idx_v[b, pl.ds(40, 16)] = zeros16

    def _gen_and_fire(r, b):
        r7 = jnp.full((16,), r * 7, dtype=jnp.int32)

        def col(j):
            return plsc.load_gather(prop_v, [r7 + j])

        x0, y0, x1, y1 = col(1), col(2), col(3), col(4)
        area = (x1 - x0) * (y1 - y0)
        one = jnp.full((16,), 1, dtype=jnp.int32)
        zero = jnp.full((16,), 0, dtype=jnp.int32)
        lvl = (jnp.where(area > 2304.0, one, zero)
               + jnp.where(area > 9216.0, one, zero)
               + jnp.where(area > 36864.0, one, zero))
        scale = jnp.where(lvl == 0, 0.25,
                          jnp.where(lvl == 1, 0.125,
                                    jnp.where(lvl == 2, 0.0625, 0.03125)))
        off = jnp.where(lvl == 0, 0,
                        jnp.where(lvl == 1, 16384,
                                  jnp.where(lvl == 2, 20480, 21504))).astype(jnp.int32)
        wl = jnp.where(lvl == 0, 128,
                       jnp.where(lvl == 1, 64,
                                 jnp.where(lvl == 2, 32, 16))).astype(jnp.int32)
        x0s = x0 * scale
        y0s = y0 * scale
        bw = (x1 * scale - x0s) / 7.0
        bh = (y1 * scale - y0s) / 7.0
        wmax = wl - 2
        bb = jnp.full((16,), b, dtype=jnp.int32)

        for j in range(4):
            p = iota + (16 * j)
            pyi = (p * 9363) >> 16
            pxi = p - pyi * 7
            yy = y0s + (pyi.astype(jnp.float32) + 0.5) * bh - 0.5
            xx = x0s + (pxi.astype(jnp.float32) + 0.5) * bw - 0.5
            yt = yy.astype(jnp.int32)
            yfi = jnp.where(yt.astype(jnp.float32) > yy, yt - 1, yt)
            xt = xx.astype(jnp.int32)
            xfi = jnp.where(xt.astype(jnp.float32) > xx, xt - 1, xt)
            ly = yy - yfi.astype(jnp.float32)
            lx = xx - xfi.astype(jnp.float32)
            hy = 1.0 - ly
            hx = 1.0 - lx
            bym = jnp.clip(yfi, 0, wmax)
            bxm = jnp.clip(xfi, 0, wmax)
            wy1 = jnp.clip(yy - bym.astype(jnp.float32), 0.0, 1.0)
            wx1 = jnp.clip(xx - bxm.astype(jnp.float32), 0.0, 1.0)
            wy0 = 1.0 - wy1
            wx0 = 1.0 - wx1
            idxp = off + bym * wl + bxm
            w4 = (wy0 * wx0, wy0 * wx1, wy1 * wx0, wy1 * wx1)
            msk = p < NSAMP
            plsc.store_scatter(idx_v, [bb, p], idxp, mask=msk)
            for c4 in range(4):
                plsc.store_scatter(w_v, [bb, p, jnp.full((16,), c4, dtype=jnp.int32)],
                                   w4[c4], mask=msk)
        pltpu.async_copy(table.at[idx_v.at[b]], rows_v.at[b], sem)

    def _wait_gather(b):
        pltpu.make_async_copy(table.at[idx_v.at[b]], rows_v.at[b], sem).wait()

    def _combine(b):
        def py_body(py, c0):
            def px_body(px, c1):
                s = py * CROP + px
                wrow = w_v[b, s, :]
                w00 = wrow[0]
                w01 = wrow[1]
                w10 = wrow[2]
                w11 = wrow[3]

                base_e = (2 * iota) * NSAMP + s
                base_o = (2 * iota + 1) * NSAMP + s
                for cb in range(8):
                    o = cb * 32

                    def corner(k):
                        x = rows_v[b, s, pl.ds(k * C + o, 32)]
                        return plsc.unpack(x, format=plsc.PackFormat.INTERLEAVED)
                    e0, o0 = corner(0)
                    e1, o1 = corner(1)
                    e2, o2 = corner(2)
                    e3, o3 = corner(3)
                    acc_e = (e0 * w00 + e1 * w01) + (e2 * w10 + e3 * w11)
                    acc_o = (o0 * w00 + o1 * w01) + (o2 * w10 + o3 * w11)
                    pos_e = base_e + (cb * 32 * NSAMP)
                    pos_o = base_o + (cb * 32 * NSAMP)
                    plsc.store_scatter(out_v, [pos_e >> 7, pos_e & 127], acc_e)
                    plsc.store_scatter(out_v, [pos_o >> 7, pos_o & 127], acc_o)
                return c1
            return lax.fori_loop(0, CROP, px_body, c0)
        lax.fori_loop(0, CROP, py_body, 0)

    @pl.when(base_roi < N_ROI)
    def _():
        _gen_and_fire(0, 0)

    def pipe_body(i, carry):
        for half in range(2):
            r = 2 * i + half
            roi = base_roi + r

            @pl.when((r + 1 < ROIS_PER_W) & (roi + 1 < N_ROI))
            def _():
                _gen_and_fire(r + 1, 1 - half)

            @pl.when(roi < N_ROI)
            def _():
                _wait_gather(half)
                _combine(half)
                pltpu.sync_copy(out_v, out.at[pl.ds(roi * OUT_ROWS, OUT_ROWS)])
        return carry

    lax.fori_loop(0, ROIS_PER_W // 2, pipe_body, 0)


def kernel(f2, f3, f4, f5, proposals):
    parts = []
    for f in (f2, f3, f4, f5):
        t = jnp.transpose(f[0], (1, 2, 0))
        tp = jnp.pad(t, ((0, 1), (0, 1), (0, 0)))
        h, w = t.shape[0], t.shape[1]
        parts.append(jnp.concatenate(
            [tp[:h, :w], tp[:h, 1:w + 1], tp[1:h + 1, :w], tp[1:h + 1, 1:w + 1]],
            axis=2).reshape(-1, PATCH).astype(jnp.bfloat16))
    table = jnp.concatenate(parts, axis=0)
    prop = jnp.pad(proposals, ((0, NW * ROIS_PER_W - proposals.shape[0]), (0, 0))).reshape(-1)
    mesh = plsc.VectorSubcoreMesh(core_axis_name="c", subcore_axis_name="s")
    k = pl.kernel(
        _sc_body,
        out_type=jax.ShapeDtypeStruct((N_ROI * OUT_ROWS, 128), jnp.float32),
        mesh=mesh,
        scratch_types=[
            pltpu.VMEM((ROIS_PER_W * 7,), jnp.float32),
            pltpu.VMEM((2, NSAMP_PAD), jnp.int32),
            pltpu.VMEM((2, NSAMP, 16), jnp.float32),
            pltpu.VMEM((2, NSAMP_PAD, PATCH), jnp.bfloat16),
            pltpu.VMEM((OUT_ROWS, 128), jnp.float32),
            pltpu.SemaphoreType.DMA,
        ],
        compiler_params=pltpu.CompilerParams(use_tc_tiling_on_sc=False,
                                             needs_layout_passes=False),
    )
    return jnp.reshape(k(table, prop), (N_ROI, C, CROP, CROP))


# speedup vs baseline: 1.0118x; 1.0118x over previous
import jax
import jax.numpy as jnp
from jax import lax
from jax.experimental import pallas as pl
from jax.experimental.pallas import tpu as pltpu
from jax.experimental.pallas import tpu_sc as plsc

CROP = 7
NSAMP = CROP * CROP
HALF = 98
HALF_PAD = 104
C = 256
N_ROI = 1000
NW = 32
ROIS_PER_W = 32
OUT_ROWS = NSAMP * C // 128


def _sc_body(table, prop, out, prop_v, idx_v, w_v, rows_v, out_v, sem):
    wid = lax.axis_index("s") * 2 + lax.axis_index("c")
    base_roi = wid * ROIS_PER_W
    pltpu.sync_copy(prop.at[pl.ds(base_roi * 7, ROIS_PER_W * 7)], prop_v)

    iota = lax.iota(jnp.int32, 16)
    zeros16 = jnp.zeros((16,), dtype=jnp.int32)
    for b in range(2):
        idx_v[b, 0, pl.ds(88, 16)] = zeros16
        idx_v[b, 1, pl.ds(88, 16)] = zeros16

    def _gen_and_fire(r, b):
        r7 = jnp.full((16,), r * 7, dtype=jnp.int32)

        def col(j):
            return plsc.load_gather(prop_v, [r7 + j])

        x0, y0, x1, y1 = col(1), col(2), col(3), col(4)
        area = (x1 - x0) * (y1 - y0)
        one = jnp.full((16,), 1, dtype=jnp.int32)
        zero = jnp.full((16,), 0, dtype=jnp.int32)
        lvl = (jnp.where(area > 2304.0, one, zero)
               + jnp.where(area > 9216.0, one, zero)
               + jnp.where(area > 36864.0, one, zero))
        scale = jnp.where(lvl == 0, 0.25,
                          jnp.where(lvl == 1, 0.125,
                                    jnp.where(lvl == 2, 0.0625, 0.03125)))
        off = jnp.where(lvl == 0, 0,
                        jnp.where(lvl == 1, 16384,
                                  jnp.where(lvl == 2, 20480, 21504))).astype(jnp.int32)
        wl = jnp.where(lvl == 0, 128,
                       jnp.where(lvl == 1, 64,
                                 jnp.where(lvl == 2, 32, 16))).astype(jnp.int32)
        x0s = x0 * scale
        y0s = y0 * scale
        bw = (x1 * scale - x0s) / 7.0
        bh = (y1 * scale - y0s) / 7.0
        wmax = wl - 1
        bb = jnp.full((16,), b, dtype=jnp.int32)

        for j in range(4):
            p = iota + (16 * j)
            pyi = (p * 9363) >> 16
            pxi = p - pyi * 7
            yy = y0s + (pyi.astype(jnp.float32) + 0.5) * bh - 0.5
            xx = x0s + (pxi.astype(jnp.float32) + 0.5) * bw - 0.5
            yt = yy.astype(jnp.int32)
            yfi = jnp.where(yt.astype(jnp.float32) > yy, yt - 1, yt)
            xt = xx.astype(jnp.int32)
            xfi = jnp.where(xt.astype(jnp.float32) > xx, xt - 1, xt)
            ly = yy - yfi.astype(jnp.float32)
            lx = xx - xfi.astype(jnp.float32)
            hy = 1.0 - ly
            hx = 1.0 - lx
            y0c = jnp.clip(yfi, 0, wmax)
            y1c = jnp.clip(yfi + 1, 0, wmax)
            x0c = jnp.clip(xfi, 0, wmax)
            x1c = jnp.clip(xfi + 1, 0, wmax)
            b0 = off + y0c * wl
            b1 = off + y1c * wl
            idx4 = (b0 + x0c, b0 + x1c, b1 + x0c, b1 + x1c)
            w4 = (hy * hx, hy * lx, ly * hx, ly * lx)
            msk = p < NSAMP
            pos = p * 4
            for c4 in range(4):
                posc = pos + c4
                g = (posc * 669) >> 16
                rem = posc - g * HALF
                plsc.store_scatter(idx_v, [bb, g, rem], idx4[c4], mask=msk)
                plsc.store_scatter(w_v, [bb, p, jnp.full((16,), c4, dtype=jnp.int32)],
                                   w4[c4], mask=msk)
        for g in range(2):
            pltpu.async_copy(table.at[idx_v.at[b, g]],
                             rows_v.at[b, pl.ds(g * HALF_PAD, HALF_PAD)], sem)

    def _wait_gather(b):
        for g in range(2):
            pltpu.make_async_copy(table.at[idx_v.at[b, g]],
                                  rows_v.at[b, pl.ds(g * HALF_PAD, HALF_PAD)],
                                  sem).wait()

    def _combine(b):
        def py_body(py, c0):
            def px_body(px, c1):
                s = py * CROP + px
                s4 = s * 4
                wrow = w_v[b, s, :]
                w00 = wrow[0]
                w01 = wrow[1]
                w10 = wrow[2]
                w11 = wrow[3]

                def row(c4):
                    posc = s4 + c4
                    return posc + 6 * ((posc * 669) >> 16)
                r0, r1, r2, r3 = row(0), row(1), row(2), row(3)
                base = iota * NSAMP + s
                for cb in range(16):
                    sl = pl.ds(cb * 16, 16)
                    acc = ((rows_v[b, r0, sl] * w00 + rows_v[b, r1, sl] * w01)
                           + (rows_v[b, r2, sl] * w10 + rows_v[b, r3, sl] * w11))
                    pos = base + (cb * 16 * NSAMP)
                    plsc.store_scatter(out_v, [pos >> 7, pos & 127], acc)
                return c1
            return lax.fori_loop(0, CROP, px_body, c0)
        lax.fori_loop(0, CROP, py_body, 0)

    @pl.when(base_roi < N_ROI)
    def _():
        _gen_and_fire(0, 0)

    def pipe_body(i, carry):
        for half in range(2):
            r = 2 * i + half
            roi = base_roi + r

            @pl.when((r + 1 < ROIS_PER_W) & (roi + 1 < N_ROI))
            def _():
                _gen_and_fire(r + 1, 1 - half)

            @pl.when(roi < N_ROI)
            def _():
                _wait_gather(half)
                _combine(half)
                pltpu.sync_copy(out_v, out.at[pl.ds(roi * OUT_ROWS, OUT_ROWS)])
        return carry

    lax.fori_loop(0, ROIS_PER_W // 2, pipe_body, 0)


def kernel(f2, f3, f4, f5, proposals):
    parts = [jnp.transpose(f[0], (1, 2, 0)).reshape(-1, C) for f in (f2, f3, f4, f5)]
    table = jnp.concatenate(parts, axis=0)
    prop = jnp.pad(proposals, ((0, NW * ROIS_PER_W - proposals.shape[0]), (0, 0))).reshape(-1)
    mesh = plsc.VectorSubcoreMesh(core_axis_name="c", subcore_axis_name="s")
    k = pl.kernel(
        _sc_body,
        out_type=jax.ShapeDtypeStruct((N_ROI * OUT_ROWS, 128), jnp.float32),
        mesh=mesh,
        scratch_types=[
            pltpu.VMEM((ROIS_PER_W * 7,), jnp.float32),
            pltpu.VMEM((2, 2, HALF_PAD), jnp.int32),
            pltpu.VMEM((2, NSAMP, 16), jnp.float32),
            pltpu.VMEM((2, 2 * HALF_PAD, C), jnp.float32),
            pltpu.VMEM((OUT_ROWS, 128), jnp.float32),
            pltpu.SemaphoreType.DMA,
        ],
        compiler_params=pltpu.CompilerParams(use_tc_tiling_on_sc=False,
                                             needs_layout_passes=False),
    )
    return jnp.reshape(k(table, prop), (N_ROI, C, CROP, CROP))
